# baseline (speedup 1.0000x reference)
import functools

import jax
import jax.numpy as jnp
from jax import lax
from jax.experimental import pallas as pl
from jax.experimental.pallas import tpu as pltpu
from jax.experimental.pallas import tpu_sc as plsc

_B = 16384
_NSH = 100
_IN = 64
_EMB = 128

_NC = 2
_NS = 16
_NW = _NC * _NS
_BPW = _B // _NW
_NGRP = _BPW // 16


def _sc_gather(shelf_hbm, inv_hbm, feat_hbm, shelf_v, rows_v, sem):
    wid = lax.axis_index("s") * _NC + lax.axis_index("c")
    base = wid * _BPW
    pltpu.sync_copy(shelf_hbm.at[pl.ds(base, _BPW)], shelf_v)
    lane = lax.iota(jnp.int32, 16)

    def issue(g):
        grp = shelf_v[pl.ds(g * 16, 16)]
        grp = jnp.where(grp == _NSH, 0, grp)
        for l in range(16):
            r = jnp.sum(jnp.where(lane == l, grp, 0))
            row = g * 16 + l
            pltpu.async_copy(inv_hbm.at[base + row, r], rows_v.at[row], sem)

    def wait16():
        for _ in range(16):
            pltpu.make_async_copy(
                inv_hbm.at[base, 0], rows_v.at[0], sem
            ).wait()

    issue(0)
    issue(1)

    def body(g, carry):
        issue(g)
        wait16()
        return carry

    lax.fori_loop(2, _NGRP, body, 0, unroll=False)
    wait16()
    wait16()
    pltpu.sync_copy(rows_v, feat_hbm.at[pl.ds(base, _BPW)])


@jax.jit
def _gather_feats(shelf, inv):
    mesh = plsc.VectorSubcoreMesh(core_axis_name="c", subcore_axis_name="s")
    f = pl.kernel(
        _sc_gather,
        mesh=mesh,
        out_type=jax.ShapeDtypeStruct((_B, _IN), jnp.float32),
        scratch_types=[
            pltpu.VMEM((_BPW,), jnp.int32),
            pltpu.VMEM((_BPW, _IN), jnp.float32),
            pltpu.SemaphoreType.DMA,
        ],
        compiler_params=pltpu.CompilerParams(needs_layout_passes=False),
    )
    return f(shelf, inv)


def _lrelu(x):
    return jnp.where(x > 0, x, 0.01 * x)


def _tc_mlp(coord_b, feat_b, shelf_b, rW1, rb1, rW2, rb2,
            sW1, sb1, sW2, sb2, ph, out_b):
    c = coord_b[...]
    w1 = rW1[...]
    r1 = (c[:, 0:1] * w1[0:1, :] + c[:, 1:2] * w1[1:2, :]
          + c[:, 2:3] * w1[2:3, :]) + rb1[...]
    r1 = _lrelu(r1)
    r2 = _lrelu(jnp.dot(r1, rW2[...], preferred_element_type=jnp.float32)
                + rb2[...])
    s1 = _lrelu(jnp.dot(feat_b[...], sW1[...],
                        preferred_element_type=jnp.float32) + sb1[...])
    s2 = _lrelu(jnp.dot(s1, sW2[...], preferred_element_type=jnp.float32)
                + sb2[...])
    no_shelf = shelf_b[...] == _NSH
    out_b[...] = r2 + jnp.where(no_shelf, ph[...], s2)


def _full(shape):
    return pl.BlockSpec(shape, lambda i: tuple(0 for _ in shape))


@functools.partial(jax.jit, static_argnames=("blk",))
def _mlp(coord, feat, shelf2d, rW1, rb1, rW2, rb2, sW1, sb1, sW2, sb2,
         ph2d, blk=2048):
    grid = (_B // blk,)
    return pl.pallas_call(
        _tc_mlp,
        grid=grid,
        in_specs=[
            pl.BlockSpec((blk, 3), lambda i: (i, 0)),
            pl.BlockSpec((blk, _IN), lambda i: (i, 0)),
            pl.BlockSpec((blk, 1), lambda i: (i, 0)),
            _full((3, _EMB)), _full((1, _EMB)),
            _full((_EMB, _EMB)), _full((1, _EMB)),
            _full((_IN, _EMB)), _full((1, _EMB)),
            _full((_EMB, _EMB)), _full((1, _EMB)),
            _full((1, _EMB)),
        ],
        out_specs=pl.BlockSpec((blk, _EMB), lambda i: (i, 0)),
        out_shape=jax.ShapeDtypeStruct((_B, _EMB), jnp.float32),
    )(coord, feat, shelf2d, rW1, rb1, rW2, rb2, sW1, sb1, sW2, sb2, ph2d)


def kernel(shelf, coord, inventory, rW1, rb1, rW2, rb2,
           sW1, sb1, sW2, sb2, placeholder):
    feat = _gather_feats(shelf, inventory)
    out = _mlp(coord, feat, shelf.reshape(_B, 1),
               rW1, rb1.reshape(1, _EMB), rW2, rb2.reshape(1, _EMB),
               sW1, sb1.reshape(1, _EMB), sW2, sb2.reshape(1, _EMB),
               placeholder.reshape(1, _EMB))
    return out.reshape(_B, 1, _EMB)


# speedup vs baseline: 1.2509x; 1.2509x over previous
import functools

import jax
import jax.numpy as jnp
from jax import lax
from jax.experimental import pallas as pl
from jax.experimental.pallas import tpu as pltpu

_B = 16384
_NSH = 100
_IN = 64
_EMB = 128
_IBLK = 2048


def _lrelu(x):
    return jnp.where(x > 0, x, 0.01 * x)


def _fused(shelf_b, coord_b, invt_b, rW1, rb1, rW2, rb2,
           sW1, sb1, sW2, sb2, ph, out_b, featT):
    r = pl.program_id(1)

    @pl.when(r == 0)
    def _init():
        featT[...] = jnp.zeros_like(featT)

    mask = shelf_b[...] == r
    featT[...] += jnp.where(mask, invt_b[0], 0.0)

    @pl.when(r == _NSH - 1)
    def _finish():
        c = coord_b[...]
        w1 = rW1[...]
        r1 = (c[:, 0:1] * w1[0:1, :] + c[:, 1:2] * w1[1:2, :]
              + c[:, 2:3] * w1[2:3, :]) + rb1[...]
        r1 = _lrelu(r1)
        r2 = _lrelu(jnp.dot(r1, rW2[...], preferred_element_type=jnp.float32)
                    + rb2[...])
        s1 = lax.dot_general(featT[...], sW1[...],
                             dimension_numbers=(((0,), (0,)), ((), ())),
                             preferred_element_type=jnp.float32)
        s1 = _lrelu(s1 + sb1[...])
        s2 = _lrelu(jnp.dot(s1, sW2[...],
                            preferred_element_type=jnp.float32) + sb2[...])
        no_shelf = shelf_b[...].reshape(_IBLK, 1) == _NSH
        out_b[...] = r2 + jnp.where(no_shelf, ph[...], s2)


def _full(shape):
    return pl.BlockSpec(shape, lambda i, r: tuple(0 for _ in shape))


@jax.jit
def _run(shelf2d, coord, inv_t, rW1, rb1, rW2, rb2, sW1, sb1, sW2, sb2,
         ph2d):
    grid = (_B // _IBLK, _NSH)
    return pl.pallas_call(
        _fused,
        grid=grid,
        in_specs=[
            pl.BlockSpec((1, _IBLK), lambda i, r: (0, i)),
            pl.BlockSpec((_IBLK, 3), lambda i, r: (i, 0)),
            pl.BlockSpec((1, _IN, _IBLK), lambda i, r: (r, 0, i)),
            _full((3, _EMB)), _full((1, _EMB)),
            _full((_EMB, _EMB)), _full((1, _EMB)),
            _full((_IN, _EMB)), _full((1, _EMB)),
            _full((_EMB, _EMB)), _full((1, _EMB)),
            _full((1, _EMB)),
        ],
        out_specs=pl.BlockSpec((_IBLK, _EMB), lambda i, r: (i, 0)),
        out_shape=jax.ShapeDtypeStruct((_B, _EMB), jnp.float32),
        scratch_shapes=[pltpu.VMEM((_IN, _IBLK), jnp.float32)],
        compiler_params=pltpu.CompilerParams(
            dimension_semantics=("parallel", "arbitrary"),
        ),
    )(shelf2d, coord, inv_t, rW1, rb1, rW2, rb2, sW1, sb1, sW2, sb2, ph2d)


def kernel(shelf, coord, inventory, rW1, rb1, rW2, rb2,
           sW1, sb1, sW2, sb2, placeholder):
    inv_t = jnp.transpose(inventory, (1, 2, 0))
    out = _run(shelf.reshape(1, _B), coord, inv_t,
               rW1, rb1.reshape(1, _EMB), rW2, rb2.reshape(1, _EMB),
               sW1, sb1.reshape(1, _EMB), sW2, sb2.reshape(1, _EMB),
               placeholder.reshape(1, _EMB))
    return out.reshape(_B, 1, _EMB)


# speedup vs baseline: 2.9507x; 2.3588x over previous
import functools

import jax
import jax.numpy as jnp
from jax import lax
from jax.experimental import pallas as pl
from jax.experimental.pallas import tpu as pltpu

_B = 16384
_NSH = 100
_IN = 64
_EMB = 128
_IBLK = 2048
_RBLK = 4
_NRB = _NSH // _RBLK


def _lrelu(x):
    return jnp.where(x > 0, x, 0.01 * x)


def _fused(shelf_b, coord_b, invt_b, rW1, rb1, rW2, rb2,
           sW1, sb1, sW2, sb2, ph, out_b, featT):
    rb = pl.program_id(1)

    @pl.when(rb == 0)
    def _init():
        featT[...] = jnp.zeros_like(featT)

    shelf_row = shelf_b[...]
    acc = featT[...]
    for j in range(_RBLK):
        acc += jnp.where(shelf_row == rb * _RBLK + j, invt_b[j], 0.0)
    featT[...] = acc

    @pl.when(rb == _NRB - 1)
    def _finish():
        c = coord_b[...]
        w1 = rW1[...]
        r1 = (c[:, 0:1] * w1[0:1, :] + c[:, 1:2] * w1[1:2, :]
              + c[:, 2:3] * w1[2:3, :]) + rb1[...]
        r1 = _lrelu(r1)
        r2 = _lrelu(jnp.dot(r1, rW2[...], preferred_element_type=jnp.float32)
                    + rb2[...])
        s1 = lax.dot_general(featT[...], sW1[...],
                             dimension_numbers=(((0,), (0,)), ((), ())),
                             preferred_element_type=jnp.float32)
        s1 = _lrelu(s1 + sb1[...])
        s2 = _lrelu(jnp.dot(s1, sW2[...],
                            preferred_element_type=jnp.float32) + sb2[...])
        no_shelf = shelf_b[...].reshape(_IBLK, 1) == _NSH
        out_b[...] = r2 + jnp.where(no_shelf, ph[...], s2)


def _full(shape):
    return pl.BlockSpec(shape, lambda i, r: tuple(0 for _ in shape))


@jax.jit
def _run(shelf2d, coord, inv_t, rW1, rb1, rW2, rb2, sW1, sb1, sW2, sb2,
         ph2d):
    grid = (_B // _IBLK, _NRB)
    return pl.pallas_call(
        _fused,
        grid=grid,
        in_specs=[
            pl.BlockSpec((1, _IBLK), lambda i, r: (0, i)),
            pl.BlockSpec((_IBLK, 3), lambda i, r: (i, 0)),
            pl.BlockSpec((_RBLK, _IN, _IBLK), lambda i, r: (r, 0, i)),
            _full((3, _EMB)), _full((1, _EMB)),
            _full((_EMB, _EMB)), _full((1, _EMB)),
            _full((_IN, _EMB)), _full((1, _EMB)),
            _full((_EMB, _EMB)), _full((1, _EMB)),
            _full((1, _EMB)),
        ],
        out_specs=pl.BlockSpec((_IBLK, _EMB), lambda i, r: (i, 0)),
        out_shape=jax.ShapeDtypeStruct((_B, _EMB), jnp.float32),
        scratch_shapes=[pltpu.VMEM((_IN, _IBLK), jnp.float32)],
        compiler_params=pltpu.CompilerParams(
            dimension_semantics=("parallel", "arbitrary"),
        ),
    )(shelf2d, coord, inv_t, rW1, rb1, rW2, rb2, sW1, sb1, sW2, sb2, ph2d)


def kernel(shelf, coord, inventory, rW1, rb1, rW2, rb2,
           sW1, sb1, sW2, sb2, placeholder):
    inv_t = jnp.transpose(inventory, (1, 2, 0))
    out = _run(shelf.reshape(1, _B), coord, inv_t,
               rW1, rb1.reshape(1, _EMB), rW2, rb2.reshape(1, _EMB),
               sW1, sb1.reshape(1, _EMB), sW2, sb2.reshape(1, _EMB),
               placeholder.reshape(1, _EMB))
    return out.reshape(_B, 1, _EMB)


# speedup vs baseline: 4.1448x; 1.4047x over previous
import functools

import jax
import jax.numpy as jnp
from jax import lax
from jax.experimental import pallas as pl
from jax.experimental.pallas import tpu as pltpu

_B = 16384
_NSH = 100
_IN = 64
_EMB = 128
_IBLK = 2048
_RBLK = 10
_NRB = _NSH // _RBLK


def _lrelu(x):
    return jnp.where(x > 0, x, 0.01 * x)


def _fused(shelf_b, coord_b, invt_b, rW1, rb1, rW2, rb2,
           sW1, sb1, sW2, sb2, ph, out_b, featT):
    rb = pl.program_id(1)

    @pl.when(rb == 0)
    def _init():
        featT[...] = jnp.zeros_like(featT)

    shelf_row = shelf_b[...]
    acc = featT[...]
    for j in range(_RBLK):
        acc += jnp.where(shelf_row == rb * _RBLK + j, invt_b[j], 0.0)
    featT[...] = acc

    @pl.when(rb == _NRB - 1)
    def _finish():
        c = coord_b[...]
        w1 = rW1[...]
        r1 = (c[:, 0:1] * w1[0:1, :] + c[:, 1:2] * w1[1:2, :]
              + c[:, 2:3] * w1[2:3, :]) + rb1[...]
        r1 = _lrelu(r1)
        r2 = _lrelu(jnp.dot(r1, rW2[...], preferred_element_type=jnp.float32)
                    + rb2[...])
        s1 = lax.dot_general(featT[...], sW1[...],
                             dimension_numbers=(((0,), (0,)), ((), ())),
                             preferred_element_type=jnp.float32)
        s1 = _lrelu(s1 + sb1[...])
        s2 = _lrelu(jnp.dot(s1, sW2[...],
                            preferred_element_type=jnp.float32) + sb2[...])
        no_shelf = shelf_b[...].reshape(_IBLK, 1) == _NSH
        out_b[...] = r2 + jnp.where(no_shelf, ph[...], s2)


def _full(shape):
    return pl.BlockSpec(shape, lambda i, r: tuple(0 for _ in shape))


@jax.jit
def _run(shelf2d, coord, inv_t, rW1, rb1, rW2, rb2, sW1, sb1, sW2, sb2,
         ph2d):
    grid = (_B // _IBLK, _NRB)
    return pl.pallas_call(
        _fused,
        grid=grid,
        in_specs=[
            pl.BlockSpec((1, _IBLK), lambda i, r: (0, i)),
            pl.BlockSpec((_IBLK, 3), lambda i, r: (i, 0)),
            pl.BlockSpec((_RBLK, _IN, _IBLK), lambda i, r: (r, 0, i)),
            _full((3, _EMB)), _full((1, _EMB)),
            _full((_EMB, _EMB)), _full((1, _EMB)),
            _full((_IN, _EMB)), _full((1, _EMB)),
            _full((_EMB, _EMB)), _full((1, _EMB)),
            _full((1, _EMB)),
        ],
        out_specs=pl.BlockSpec((_IBLK, _EMB), lambda i, r: (i, 0)),
        out_shape=jax.ShapeDtypeStruct((_B, _EMB), jnp.float32),
        scratch_shapes=[pltpu.VMEM((_IN, _IBLK), jnp.float32)],
        compiler_params=pltpu.CompilerParams(
            dimension_semantics=("parallel", "arbitrary"),
        ),
    )(shelf2d, coord, inv_t, rW1, rb1, rW2, rb2, sW1, sb1, sW2, sb2, ph2d)


def kernel(shelf, coord, inventory, rW1, rb1, rW2, rb2,
           sW1, sb1, sW2, sb2, placeholder):
    inv_t = jnp.transpose(inventory, (1, 2, 0))
    out = _run(shelf.reshape(1, _B), coord, inv_t,
               rW1, rb1.reshape(1, _EMB), rW2, rb2.reshape(1, _EMB),
               sW1, sb1.reshape(1, _EMB), sW2, sb2.reshape(1, _EMB),
               placeholder.reshape(1, _EMB))
    return out.reshape(_B, 1, _EMB)


# speedup vs baseline: 4.5454x; 1.0967x over previous
import functools

import jax
import jax.numpy as jnp
from jax import lax
from jax.experimental import pallas as pl
from jax.experimental.pallas import tpu as pltpu

_B = 16384
_NSH = 100
_IN = 64
_EMB = 128
_IBLK = 2048
_RBLK = 25
_NRB = _NSH // _RBLK


def _lrelu(x):
    return jnp.where(x > 0, x, 0.01 * x)


def _fused(shelf_b, coord_b, invt_b, rW1, rb1, rW2, rb2,
           sW1, sb1, sW2, sb2, ph, out_b, featT):
    rb = pl.program_id(1)

    @pl.when(rb == 0)
    def _init():
        featT[...] = jnp.zeros_like(featT)

    shelf_row = shelf_b[...]
    acc = featT[...]
    for j in range(_RBLK):
        acc += jnp.where(shelf_row == rb * _RBLK + j, invt_b[j], 0.0)
    featT[...] = acc

    @pl.when(rb == _NRB - 1)
    def _finish():
        c = coord_b[...]
        w1 = rW1[...]
        r1 = (c[:, 0:1] * w1[0:1, :] + c[:, 1:2] * w1[1:2, :]
              + c[:, 2:3] * w1[2:3, :]) + rb1[...]
        r1 = _lrelu(r1)
        r2 = _lrelu(jnp.dot(r1, rW2[...], preferred_element_type=jnp.float32)
                    + rb2[...])
        s1 = lax.dot_general(featT[...], sW1[...],
                             dimension_numbers=(((0,), (0,)), ((), ())),
                             preferred_element_type=jnp.float32)
        s1 = _lrelu(s1 + sb1[...])
        s2 = _lrelu(jnp.dot(s1, sW2[...],
                            preferred_element_type=jnp.float32) + sb2[...])
        no_shelf = shelf_b[...].reshape(_IBLK, 1) == _NSH
        out_b[...] = r2 + jnp.where(no_shelf, ph[...], s2)


def _full(shape):
    return pl.BlockSpec(shape, lambda i, r: tuple(0 for _ in shape))


@jax.jit
def _run(shelf2d, coord, inv_t, rW1, rb1, rW2, rb2, sW1, sb1, sW2, sb2,
         ph2d):
    grid = (_B // _IBLK, _NRB)
    return pl.pallas_call(
        _fused,
        grid=grid,
        in_specs=[
            pl.BlockSpec((1, _IBLK), lambda i, r: (0, i)),
            pl.BlockSpec((_IBLK, 3), lambda i, r: (i, 0)),
            pl.BlockSpec((_RBLK, _IN, _IBLK), lambda i, r: (r, 0, i)),
            _full((3, _EMB)), _full((1, _EMB)),
            _full((_EMB, _EMB)), _full((1, _EMB)),
            _full((_IN, _EMB)), _full((1, _EMB)),
            _full((_EMB, _EMB)), _full((1, _EMB)),
            _full((1, _EMB)),
        ],
        out_specs=pl.BlockSpec((_IBLK, _EMB), lambda i, r: (i, 0)),
        out_shape=jax.ShapeDtypeStruct((_B, _EMB), jnp.float32),
        scratch_shapes=[pltpu.VMEM((_IN, _IBLK), jnp.float32)],
        compiler_params=pltpu.CompilerParams(
            dimension_semantics=("parallel", "arbitrary"),
        ),
    )(shelf2d, coord, inv_t, rW1, rb1, rW2, rb2, sW1, sb1, sW2, sb2, ph2d)


def kernel(shelf, coord, inventory, rW1, rb1, rW2, rb2,
           sW1, sb1, sW2, sb2, placeholder):
    inv_t = jnp.transpose(inventory, (1, 2, 0))
    out = _run(shelf.reshape(1, _B), coord, inv_t,
               rW1, rb1.reshape(1, _EMB), rW2, rb2.reshape(1, _EMB),
               sW1, sb1.reshape(1, _EMB), sW2, sb2.reshape(1, _EMB),
               placeholder.reshape(1, _EMB))
    return out.reshape(_B, 1, _EMB)


# speedup vs baseline: 4.6957x; 1.0331x over previous
import functools

import jax
import jax.numpy as jnp
from jax import lax
from jax.experimental import pallas as pl
from jax.experimental.pallas import tpu as pltpu

_B = 16384
_NSH = 100
_IN = 64
_EMB = 128
_IBLK = 2048
_RBLK = 25
_NRB = _NSH // _RBLK


def _lrelu(x):
    return jnp.where(x > 0, x, 0.01 * x)


def _fused(shelf_b, coord_b, invt_b, rW1, rb1, rW2, rb2,
           sW1, sb1, sW2, sb2, ph, out_b, featT):
    rb = pl.program_id(1)

    @pl.when(rb == 0)
    def _init():
        featT[...] = jnp.zeros_like(featT)

    shelf_row = shelf_b[...]
    acc = featT[...]
    for j in range(_RBLK):
        acc += jnp.where(shelf_row == rb * _RBLK + j, invt_b[j], 0.0)
    featT[...] = acc

    @pl.when(rb == _NRB - 1)
    def _finish():
        r1 = _lrelu(jnp.dot(coord_b[...], rW1[...],
                            preferred_element_type=jnp.float32) + rb1[...])
        r2 = _lrelu(jnp.dot(r1, rW2[...], preferred_element_type=jnp.float32)
                    + rb2[...])
        s1 = lax.dot_general(featT[...], sW1[...],
                             dimension_numbers=(((0,), (0,)), ((), ())),
                             preferred_element_type=jnp.float32)
        s1 = _lrelu(s1 + sb1[...])
        s2 = _lrelu(jnp.dot(s1, sW2[...],
                            preferred_element_type=jnp.float32) + sb2[...])
        no_shelf = shelf_b[...].reshape(_IBLK, 1) == _NSH
        out_b[...] = r2 + jnp.where(no_shelf, ph[...], s2)


def _full(shape):
    return pl.BlockSpec(shape, lambda i, r: tuple(0 for _ in shape))


@jax.jit
def _run(shelf2d, coord, inv_t, rW1, rb1, rW2, rb2, sW1, sb1, sW2, sb2,
         ph2d):
    grid = (_B // _IBLK, _NRB)
    return pl.pallas_call(
        _fused,
        grid=grid,
        in_specs=[
            pl.BlockSpec((1, _IBLK), lambda i, r: (0, i)),
            pl.BlockSpec((_IBLK, 3), lambda i, r: (i, 0)),
            pl.BlockSpec((_RBLK, _IN, _IBLK), lambda i, r: (r, 0, i)),
            _full((3, _EMB)), _full((1, _EMB)),
            _full((_EMB, _EMB)), _full((1, _EMB)),
            _full((_IN, _EMB)), _full((1, _EMB)),
            _full((_EMB, _EMB)), _full((1, _EMB)),
            _full((1, _EMB)),
        ],
        out_specs=pl.BlockSpec((_IBLK, _EMB), lambda i, r: (i, 0)),
        out_shape=jax.ShapeDtypeStruct((_B, _EMB), jnp.float32),
        scratch_shapes=[pltpu.VMEM((_IN, _IBLK), jnp.float32)],
        compiler_params=pltpu.CompilerParams(
            dimension_semantics=("parallel", "arbitrary"),
        ),
    )(shelf2d, coord, inv_t, rW1, rb1, rW2, rb2, sW1, sb1, sW2, sb2, ph2d)


def kernel(shelf, coord, inventory, rW1, rb1, rW2, rb2,
           sW1, sb1, sW2, sb2, placeholder):
    inv_t = jnp.transpose(inventory, (1, 2, 0))
    out = _run(shelf.reshape(1, _B), coord, inv_t,
               rW1, rb1.reshape(1, _EMB), rW2, rb2.reshape(1, _EMB),
               sW1, sb1.reshape(1, _EMB), sW2, sb2.reshape(1, _EMB),
               placeholder.reshape(1, _EMB))
    return out.reshape(_B, 1, _EMB)


# speedup vs baseline: 4.7043x; 1.0018x over previous
import jax
import jax.numpy as jnp
from jax import lax
from jax.experimental import pallas as pl
from jax.experimental.pallas import tpu as pltpu

_B = 16384
_NSH = 100
_IN = 64
_EMB = 128
_IBLK = 2048
_RBLK = 25
_NRB = _NSH // _RBLK


def _lrelu(x):
    return jnp.where(x > 0, x, 0.01 * x)


def _fused(shelf_b, coord_b, invt_b, rW1, rb1, rW2, rb2,
           sW1, sb1, sW2, sb2, ph, out_b, featT):
    rb = pl.program_id(1)

    @pl.when(rb == 0)
    def _init():
        featT[...] = jnp.zeros_like(featT)

    shelf_row = shelf_b[...]
    acc = featT[...]
    for j in range(_RBLK):
        acc += jnp.where(shelf_row == rb * _RBLK + j, invt_b[j], 0.0)
    featT[...] = acc

    @pl.when(rb == _NRB - 1)
    def _finish():
        r1 = _lrelu(jnp.dot(coord_b[...], rW1[...],
                            preferred_element_type=jnp.float32) + rb1[...])
        r2 = _lrelu(jnp.dot(r1, rW2[...], preferred_element_type=jnp.float32)
                    + rb2[...])
        s1 = lax.dot_general(featT[...], sW1[...],
                             dimension_numbers=(((0,), (0,)), ((), ())),
                             preferred_element_type=jnp.float32)
        s1 = _lrelu(s1 + sb1[...])
        s2 = _lrelu(jnp.dot(s1, sW2[...],
                            preferred_element_type=jnp.float32) + sb2[...])
        no_shelf = shelf_b[...].reshape(_IBLK, 1) == _NSH
        out_b[...] = r2 + jnp.where(no_shelf, ph[...], s2)


def _full(shape):
    return pl.BlockSpec(shape, lambda i, r: tuple(0 for _ in shape))


@jax.jit
def _run(shelf2d, coord, inv_t, rW1, rb1, rW2, rb2, sW1, sb1, sW2, sb2,
         ph2d):
    grid = (_B // _IBLK, _NRB)
    return pl.pallas_call(
        _fused,
        grid=grid,
        in_specs=[
            pl.BlockSpec((1, _IBLK), lambda i, r: (0, i)),
            pl.BlockSpec((_IBLK, 3), lambda i, r: (i, 0)),
            pl.BlockSpec((_RBLK, _IN, _IBLK), lambda i, r: (r, 0, i)),
            _full((3, _EMB)), _full((1, _EMB)),
            _full((_EMB, _EMB)), _full((1, _EMB)),
            _full((_IN, _EMB)), _full((1, _EMB)),
            _full((_EMB, _EMB)), _full((1, _EMB)),
            _full((1, _EMB)),
        ],
        out_specs=pl.BlockSpec((_IBLK, _EMB), lambda i, r: (i, 0)),
        out_shape=jax.ShapeDtypeStruct((_B, _EMB), jnp.float32),
        scratch_shapes=[pltpu.VMEM((_IN, _IBLK), jnp.float32)],
        compiler_params=pltpu.CompilerParams(
            dimension_semantics=("parallel", "arbitrary"),
        ),
    )(shelf2d, coord, inv_t, rW1, rb1, rW2, rb2, sW1, sb1, sW2, sb2, ph2d)


def kernel(shelf, coord, inventory, rW1, rb1, rW2, rb2,
           sW1, sb1, sW2, sb2, placeholder):
    inv_t = jnp.transpose(inventory, (1, 2, 0))
    out = _run(shelf.reshape(1, _B), coord, inv_t,
               rW1, rb1.reshape(1, _EMB), rW2, rb2.reshape(1, _EMB),
               sW1, sb1.reshape(1, _EMB), sW2, sb2.reshape(1, _EMB),
               placeholder.reshape(1, _EMB))
    return out.reshape(_B, 1, _EMB)
